# baseline (speedup 1.0000x reference)
import functools

import jax
import jax.numpy as jnp
from jax import lax
from jax.experimental import pallas as pl
from jax.experimental.pallas import tpu as pltpu
from jax.experimental.pallas import tpu_sc as plsc

N = 10000
E = 320000
DF = 128
DH = 64
LH = 32
NC = 10
NG = 64

NPAD = 10240
CH = 125
CPW = 80
RPT = NPAD // 16

_sc_mesh = plsc.VectorSubcoreMesh(core_axis_name="c", subcore_axis_name="s")


@functools.partial(
    pl.kernel,
    mesh=_sc_mesh,
    out_type=jax.ShapeDtypeStruct((2, NPAD, 16), jnp.float32),
    scratch_types=[
        pltpu.VMEM((CPW, CH), jnp.int32),
        pltpu.VMEM((CH, 16), jnp.float32),
        pltpu.VMEM_SHARED((NPAD, 16), jnp.float32),
    ],
    compiler_params=pltpu.CompilerParams(use_tc_tiling_on_sc=False),
)
def _sc_degree(dst_hbm, zeros_hbm, out_hbm, dst_buf, ones_v, acc_sh):
    c = lax.axis_index("c")
    s = lax.axis_index("s")
    w = s * 2 + c

    def fill(i, carry):
        ones_v[i, :] = jnp.ones((16,), jnp.float32)
        return carry

    lax.fori_loop(0, CH, fill, 0)
    pltpu.sync_copy(zeros_hbm.at[pl.ds(s * RPT, RPT), :],
                    acc_sh.at[pl.ds(s * RPT, RPT), :])
    pltpu.sync_copy(dst_hbm.at[pl.ds(w * CPW, CPW), :], dst_buf)
    plsc.subcore_barrier()
    for j in range(CPW):
        pltpu.sync_copy(ones_v, acc_sh.at[dst_buf.at[j]], add=True)
    plsc.subcore_barrier()
    pltpu.sync_copy(acc_sh.at[pl.ds(s * RPT, RPT), :],
                    out_hbm.at[c, pl.ds(s * RPT, RPT), :])


@functools.partial(
    pl.kernel,
    mesh=_sc_mesh,
    out_type=jax.ShapeDtypeStruct((2, NPAD, DH), jnp.float32),
    scratch_types=[
        pltpu.VMEM((CPW, CH), jnp.int32),
        pltpu.VMEM((CPW, CH), jnp.int32),
        pltpu.VMEM((CH, DH), jnp.float32),
        pltpu.VMEM((CH, DH), jnp.float32),
        pltpu.SemaphoreType.DMA,
        pltpu.SemaphoreType.DMA,
        pltpu.VMEM_SHARED((NPAD, DH), jnp.float32),
    ],
    compiler_params=pltpu.CompilerParams(use_tc_tiling_on_sc=False),
)
def _sc_scatter(ys_hbm, src_hbm, dst_hbm, zeros_hbm, out_hbm,
                src_buf, dst_buf, rows0, rows1, sem0, sem1, acc_sh):
    c = lax.axis_index("c")
    s = lax.axis_index("s")
    w = s * 2 + c

    pltpu.sync_copy(zeros_hbm.at[pl.ds(s * RPT, RPT), :],
                    acc_sh.at[pl.ds(s * RPT, RPT), :])
    pltpu.sync_copy(src_hbm.at[pl.ds(w * CPW, CPW), :], src_buf)
    pltpu.sync_copy(dst_hbm.at[pl.ds(w * CPW, CPW), :], dst_buf)
    plsc.subcore_barrier()

    rows = (rows0, rows1)
    sems = (sem0, sem1)
    descs = [None, None]
    descs[0] = pltpu.async_copy(ys_hbm.at[src_buf.at[0]], rows0, sem0)
    for j in range(CPW):
        b = j % 2
        nb = (j + 1) % 2
        if j + 1 < CPW:
            descs[nb] = pltpu.async_copy(ys_hbm.at[src_buf.at[j + 1]],
                                         rows[nb], sems[nb])
        descs[b].wait()
        pltpu.sync_copy(rows[b], acc_sh.at[dst_buf.at[j]], add=True)
    plsc.subcore_barrier()
    pltpu.sync_copy(acc_sh.at[pl.ds(s * RPT, RPT), :],
                    out_hbm.at[c, pl.ds(s * RPT, RPT), :])


_RB = 1000


def _tc_xw1_body(x_ref, w1_ref, degt_ref, ys_ref, dinv_ref):
    deg = degt_ref[:, 0:1] + degt_ref[:, 1:2] + 1.0
    dinv = lax.rsqrt(deg)
    xw = jnp.dot(x_ref[...], w1_ref[...], preferred_element_type=jnp.float32)
    ys_ref[...] = xw * dinv
    dinv_ref[...] = dinv


def _tc_layer1(acc_ref, ys_ref, dinv_ref, b1_ref, w2_ref, ys2_ref):
    agg = acc_ref[0] + acc_ref[1] + ys_ref[...]
    dinv = dinv_ref[...]
    h1 = jnp.maximum(agg * dinv + b1_ref[...], 0.0)
    ys2_ref[...] = jnp.dot(h1, w2_ref[...],
                           preferred_element_type=jnp.float32) * dinv


def _tc_layer2(acc_ref, ys_ref, dinv_ref, b2_ref, wihT_ref, bih_ref, bhh_ref,
               xg_ref):
    agg = acc_ref[0] + acc_ref[1] + ys_ref[...]
    dinv = dinv_ref[...]
    h2 = jnp.maximum(agg * dinv + b2_ref[...], 0.0)
    xg_ref[...] = (jnp.dot(h2, wihT_ref[...],
                           preferred_element_type=jnp.float32)
                   + bih_ref[...] + bhh_ref[...])


def _sigm(z):
    return 1.0 / (1.0 + jnp.exp(-z))


def _tanh(z):
    return 1.0 - 2.0 / (jnp.exp(2.0 * z) + 1.0)


def _tc_lstm_head(xg_ref, whhT_ref, batch_ref, wl_ref, bl_ref, out_ref,
                  hs_ref):
    whh = whhT_ref[...]
    lane = lax.broadcasted_iota(jnp.int32, (1, 4 * LH), 1)
    is_g = jnp.logical_and(lane >= 2 * LH, lane < 3 * LH)
    smul = jnp.where(is_g, 2.0, -1.0)
    gofs = jnp.where(is_g, 1.0, 0.0)
    gscl = jnp.where(is_g, -2.0, 1.0)

    def blockstep(k, carry):
        h, cc = carry
        xblk = xg_ref[pl.ds(k * 8, 8), :]
        hrows = []
        for j in range(8):
            xt = xblk[j:j + 1, :]
            hb = jnp.broadcast_to(jnp.transpose(h), (LH, 4 * LH))
            g = xt + jnp.sum(hb * whh, axis=0, keepdims=True)
            r = 1.0 / (1.0 + jnp.exp(g * smul))
            gates = gofs + gscl * r
            i = gates[:, 0:LH]
            f = gates[:, LH:2 * LH]
            gg = gates[:, 2 * LH:3 * LH]
            o = gates[:, 3 * LH:4 * LH]
            cc = f * cc + i * gg
            h = o * (1.0 - 2.0 / (1.0 + jnp.exp(2.0 * cc)))
            hrows.append(h)
        hs_ref[pl.ds(k * 8, 8), :] = jnp.concatenate(hrows, axis=0)
        return (h, cc)

    h0 = jnp.zeros((1, LH), jnp.float32)
    c0 = jnp.zeros((1, LH), jnp.float32)
    lax.fori_loop(0, N // 8, blockstep, (h0, c0))

    sums = jnp.zeros((NG, LH), jnp.float32)
    cnt = jnp.zeros((NG, 1), jnp.float32)
    for cb in range(N // _RB):
        hsb = hs_ref[cb * _RB:(cb + 1) * _RB, :]
        bb = batch_ref[:, cb * _RB:(cb + 1) * _RB]
        ids = lax.broadcasted_iota(jnp.int32, (NG, _RB), 0)
        oh = (bb == ids).astype(jnp.float32)
        sums = sums + jnp.dot(oh, hsb, preferred_element_type=jnp.float32)
        cnt = cnt + jnp.sum(oh, axis=1, keepdims=True)
    pooled = sums / jnp.maximum(cnt, 1.0)
    logits = jnp.dot(pooled, wl_ref[...],
                     preferred_element_type=jnp.float32) + bl_ref[...]
    m = jnp.max(logits, axis=1, keepdims=True)
    lse = jnp.log(jnp.sum(jnp.exp(logits - m), axis=1, keepdims=True))
    out_ref[...] = logits - m - lse


def kernel(x, edge_index, batch, W1, b1, W2, b2, W_ih, W_hh, b_ih, b_hh,
           Wl, bl):
    src = edge_index[0].reshape(E // CH, CH)
    dst = edge_index[1].reshape(E // CH, CH)
    zeros16 = jnp.zeros((NPAD, 16), jnp.float32)
    zeros64 = jnp.zeros((NPAD, DH), jnp.float32)

    degp = _sc_degree(dst, zeros16)
    degT = jnp.transpose(degp[:, :N, 0])

    grid = (N // _RB,)
    ys1, dinv = pl.pallas_call(
        _tc_xw1_body,
        grid=grid,
        in_specs=[
            pl.BlockSpec((_RB, DF), lambda i: (i, 0)),
            pl.BlockSpec((DF, DH), lambda i: (0, 0)),
            pl.BlockSpec((_RB, 2), lambda i: (i, 0)),
        ],
        out_specs=[
            pl.BlockSpec((_RB, DH), lambda i: (i, 0)),
            pl.BlockSpec((_RB, 1), lambda i: (i, 0)),
        ],
        out_shape=[
            jax.ShapeDtypeStruct((N, DH), jnp.float32),
            jax.ShapeDtypeStruct((N, 1), jnp.float32),
        ],
    )(x, W1, degT)

    accp1 = _sc_scatter(ys1, src, dst, zeros64)

    ys2 = pl.pallas_call(
        _tc_layer1,
        grid=grid,
        in_specs=[
            pl.BlockSpec((2, _RB, DH), lambda i: (0, i, 0)),
            pl.BlockSpec((_RB, DH), lambda i: (i, 0)),
            pl.BlockSpec((_RB, 1), lambda i: (i, 0)),
            pl.BlockSpec((1, DH), lambda i: (0, 0)),
            pl.BlockSpec((DH, DH), lambda i: (0, 0)),
        ],
        out_specs=pl.BlockSpec((_RB, DH), lambda i: (i, 0)),
        out_shape=jax.ShapeDtypeStruct((N, DH), jnp.float32),
    )(accp1[:, :N, :], ys1, dinv, b1.reshape(1, DH), W2)

    accp2 = _sc_scatter(ys2, src, dst, zeros64)

    xg = pl.pallas_call(
        _tc_layer2,
        grid=grid,
        in_specs=[
            pl.BlockSpec((2, _RB, DH), lambda i: (0, i, 0)),
            pl.BlockSpec((_RB, DH), lambda i: (i, 0)),
            pl.BlockSpec((_RB, 1), lambda i: (i, 0)),
            pl.BlockSpec((1, DH), lambda i: (0, 0)),
            pl.BlockSpec((DH, 4 * LH), lambda i: (0, 0)),
            pl.BlockSpec((1, 4 * LH), lambda i: (0, 0)),
            pl.BlockSpec((1, 4 * LH), lambda i: (0, 0)),
        ],
        out_specs=pl.BlockSpec((_RB, 4 * LH), lambda i: (i, 0)),
        out_shape=jax.ShapeDtypeStruct((N, 4 * LH), jnp.float32),
    )(accp2[:, :N, :], ys2, dinv, b2.reshape(1, DH), jnp.transpose(W_ih),
      b_ih.reshape(1, 4 * LH), b_hh.reshape(1, 4 * LH))

    out = pl.pallas_call(
        _tc_lstm_head,
        out_shape=jax.ShapeDtypeStruct((NG, NC), jnp.float32),
        scratch_shapes=[pltpu.VMEM((N, LH), jnp.float32)],
    )(xg, jnp.transpose(W_hh), batch.reshape(1, N), Wl, bl.reshape(1, NC))
    return out


# speedup vs baseline: 2.5955x; 2.5955x over previous
import functools

import jax
import jax.numpy as jnp
from jax import lax
from jax.experimental import pallas as pl
from jax.experimental.pallas import tpu as pltpu
from jax.experimental.pallas import tpu_sc as plsc

N = 10000
E = 320000
DF = 128
DH = 64
LH = 32
NC = 10
NG = 64

NPAD = 10240
CH = 125
CPW = 80
RPT = NPAD // 16

_sc_mesh = plsc.VectorSubcoreMesh(core_axis_name="c", subcore_axis_name="s")


@functools.partial(
    pl.kernel,
    mesh=_sc_mesh,
    out_type=jax.ShapeDtypeStruct((2, NPAD, 16), jnp.float32),
    scratch_types=[
        pltpu.VMEM((CPW, CH), jnp.int32),
        pltpu.VMEM((CH, 16), jnp.float32),
        pltpu.VMEM_SHARED((NPAD, 16), jnp.float32),
    ],
    compiler_params=pltpu.CompilerParams(use_tc_tiling_on_sc=False),
)
def _sc_degree(dst_hbm, zeros_hbm, out_hbm, dst_buf, ones_v, acc_sh):
    c = lax.axis_index("c")
    s = lax.axis_index("s")
    w = s * 2 + c

    def fill(i, carry):
        ones_v[i, :] = jnp.ones((16,), jnp.float32)
        return carry

    lax.fori_loop(0, CH, fill, 0)
    pltpu.sync_copy(zeros_hbm.at[pl.ds(s * RPT, RPT), :],
                    acc_sh.at[pl.ds(s * RPT, RPT), :])
    pltpu.sync_copy(dst_hbm.at[pl.ds(w * CPW, CPW), :], dst_buf)
    plsc.subcore_barrier()
    for j in range(CPW):
        pltpu.sync_copy(ones_v, acc_sh.at[dst_buf.at[j]], add=True)
    plsc.subcore_barrier()
    pltpu.sync_copy(acc_sh.at[pl.ds(s * RPT, RPT), :],
                    out_hbm.at[c, pl.ds(s * RPT, RPT), :])


@functools.partial(
    pl.kernel,
    mesh=_sc_mesh,
    out_type=jax.ShapeDtypeStruct((2, NPAD, DH), jnp.float32),
    scratch_types=[
        pltpu.VMEM((CPW, CH), jnp.int32),
        pltpu.VMEM((CPW, CH), jnp.int32),
        pltpu.VMEM((CH, DH), jnp.float32),
        pltpu.VMEM((CH, DH), jnp.float32),
        pltpu.SemaphoreType.DMA,
        pltpu.SemaphoreType.DMA,
        pltpu.VMEM_SHARED((NPAD, DH), jnp.float32),
    ],
    compiler_params=pltpu.CompilerParams(use_tc_tiling_on_sc=False),
)
def _sc_scatter(ys_hbm, src_hbm, dst_hbm, zeros_hbm, out_hbm,
                src_buf, dst_buf, rows0, rows1, sem0, sem1, acc_sh):
    c = lax.axis_index("c")
    s = lax.axis_index("s")
    w = s * 2 + c

    pltpu.sync_copy(zeros_hbm.at[pl.ds(s * RPT, RPT), :],
                    acc_sh.at[pl.ds(s * RPT, RPT), :])
    pltpu.sync_copy(src_hbm.at[pl.ds(w * CPW, CPW), :], src_buf)
    pltpu.sync_copy(dst_hbm.at[pl.ds(w * CPW, CPW), :], dst_buf)
    plsc.subcore_barrier()

    rows = (rows0, rows1)
    sems = (sem0, sem1)
    descs = [None, None]
    descs[0] = pltpu.async_copy(ys_hbm.at[src_buf.at[0]], rows0, sem0)
    for j in range(CPW):
        b = j % 2
        nb = (j + 1) % 2
        if j + 1 < CPW:
            descs[nb] = pltpu.async_copy(ys_hbm.at[src_buf.at[j + 1]],
                                         rows[nb], sems[nb])
        descs[b].wait()
        pltpu.sync_copy(rows[b], acc_sh.at[dst_buf.at[j]], add=True)
    plsc.subcore_barrier()
    pltpu.sync_copy(acc_sh.at[pl.ds(s * RPT, RPT), :],
                    out_hbm.at[c, pl.ds(s * RPT, RPT), :])


_RB = 1000


def _tc_xw1_body(x_ref, w1_ref, degt_ref, ys_ref, dinv_ref):
    deg = degt_ref[:, 0:1] + degt_ref[:, 1:2] + 1.0
    dinv = lax.rsqrt(deg)
    xw = jnp.dot(x_ref[...], w1_ref[...], preferred_element_type=jnp.float32)
    ys_ref[...] = xw * dinv
    dinv_ref[...] = dinv


def _tc_layer1(acc_ref, ys_ref, dinv_ref, b1_ref, w2_ref, ys2_ref):
    agg = acc_ref[0] + acc_ref[1] + ys_ref[...]
    dinv = dinv_ref[...]
    h1 = jnp.maximum(agg * dinv + b1_ref[...], 0.0)
    ys2_ref[...] = jnp.dot(h1, w2_ref[...],
                           preferred_element_type=jnp.float32) * dinv


def _tc_layer2(acc_ref, ys_ref, dinv_ref, b2_ref, wihT_ref, bih_ref, bhh_ref,
               xg0_ref, xg1_ref, xg2_ref, xg3_ref):
    agg = acc_ref[0] + acc_ref[1] + ys_ref[...]
    dinv = dinv_ref[...]
    h2 = jnp.maximum(agg * dinv + b2_ref[...], 0.0)
    outs = (xg0_ref, xg1_ref, xg2_ref, xg3_ref)
    for k in range(4):
        bk = bih_ref[k:k + 1, :] + bhh_ref[k:k + 1, :]
        outs[k][...] = jnp.dot(h2, wihT_ref[:, k * LH:(k + 1) * LH],
                               preferred_element_type=jnp.float32) + bk


def _sigm(z):
    return 1.0 / (1.0 + jnp.exp(-z))


def _tanh(z):
    return 1.0 - 2.0 / (jnp.exp(2.0 * z) + 1.0)


def _tc_lstm_head(xg0_ref, xg1_ref, xg2_ref, xg3_ref, wsplit_ref, batch_ref,
                  wl_ref, bl_ref, out_ref, hs_ref):
    w_i = wsplit_ref[0]
    w_f = wsplit_ref[1]
    w_g = wsplit_ref[2]
    w_o = wsplit_ref[3]

    def blockstep(k, carry):
        h, cc = carry
        x0 = xg0_ref[pl.ds(k * 8, 8), :]
        x1 = xg1_ref[pl.ds(k * 8, 8), :]
        x2 = xg2_ref[pl.ds(k * 8, 8), :]
        x3 = xg3_ref[pl.ds(k * 8, 8), :]
        hrows = []
        for j in range(8):
            i = _sigm(x0[j:j + 1, :] +
                      jnp.dot(h, w_i, preferred_element_type=jnp.float32))
            f = _sigm(x1[j:j + 1, :] +
                      jnp.dot(h, w_f, preferred_element_type=jnp.float32))
            gg = _tanh(x2[j:j + 1, :] +
                       jnp.dot(h, w_g, preferred_element_type=jnp.float32))
            o = _sigm(x3[j:j + 1, :] +
                      jnp.dot(h, w_o, preferred_element_type=jnp.float32))
            cc = f * cc + i * gg
            h = o * _tanh(cc)
            hrows.append(h)
        hs_ref[pl.ds(k * 8, 8), :] = jnp.concatenate(hrows, axis=0)
        return (h, cc)

    h0 = jnp.zeros((1, LH), jnp.float32)
    c0 = jnp.zeros((1, LH), jnp.float32)
    lax.fori_loop(0, N // 8, blockstep, (h0, c0))

    sums = jnp.zeros((NG, LH), jnp.float32)
    cnt = jnp.zeros((NG, 1), jnp.float32)
    for cb in range(N // _RB):
        hsb = hs_ref[cb * _RB:(cb + 1) * _RB, :]
        bb = batch_ref[:, cb * _RB:(cb + 1) * _RB]
        ids = lax.broadcasted_iota(jnp.int32, (NG, _RB), 0)
        oh = (bb == ids).astype(jnp.float32)
        sums = sums + jnp.dot(oh, hsb, preferred_element_type=jnp.float32)
        cnt = cnt + jnp.sum(oh, axis=1, keepdims=True)
    pooled = sums / jnp.maximum(cnt, 1.0)
    logits = jnp.dot(pooled, wl_ref[...],
                     preferred_element_type=jnp.float32) + bl_ref[...]
    m = jnp.max(logits, axis=1, keepdims=True)
    lse = jnp.log(jnp.sum(jnp.exp(logits - m), axis=1, keepdims=True))
    out_ref[...] = logits - m - lse


def kernel(x, edge_index, batch, W1, b1, W2, b2, W_ih, W_hh, b_ih, b_hh,
           Wl, bl):
    src = edge_index[0].reshape(E // CH, CH)
    dst = edge_index[1].reshape(E // CH, CH)
    zeros16 = jnp.zeros((NPAD, 16), jnp.float32)
    zeros64 = jnp.zeros((NPAD, DH), jnp.float32)

    degp = _sc_degree(dst, zeros16)
    degT = jnp.transpose(degp[:, :N, 0])

    grid = (N // _RB,)
    ys1, dinv = pl.pallas_call(
        _tc_xw1_body,
        grid=grid,
        in_specs=[
            pl.BlockSpec((_RB, DF), lambda i: (i, 0)),
            pl.BlockSpec((DF, DH), lambda i: (0, 0)),
            pl.BlockSpec((_RB, 2), lambda i: (i, 0)),
        ],
        out_specs=[
            pl.BlockSpec((_RB, DH), lambda i: (i, 0)),
            pl.BlockSpec((_RB, 1), lambda i: (i, 0)),
        ],
        out_shape=[
            jax.ShapeDtypeStruct((N, DH), jnp.float32),
            jax.ShapeDtypeStruct((N, 1), jnp.float32),
        ],
    )(x, W1, degT)

    accp1 = _sc_scatter(ys1, src, dst, zeros64)

    ys2 = pl.pallas_call(
        _tc_layer1,
        grid=grid,
        in_specs=[
            pl.BlockSpec((2, _RB, DH), lambda i: (0, i, 0)),
            pl.BlockSpec((_RB, DH), lambda i: (i, 0)),
            pl.BlockSpec((_RB, 1), lambda i: (i, 0)),
            pl.BlockSpec((1, DH), lambda i: (0, 0)),
            pl.BlockSpec((DH, DH), lambda i: (0, 0)),
        ],
        out_specs=pl.BlockSpec((_RB, DH), lambda i: (i, 0)),
        out_shape=jax.ShapeDtypeStruct((N, DH), jnp.float32),
    )(accp1[:, :N, :], ys1, dinv, b1.reshape(1, DH), W2)

    accp2 = _sc_scatter(ys2, src, dst, zeros64)

    xgs = pl.pallas_call(
        _tc_layer2,
        grid=grid,
        in_specs=[
            pl.BlockSpec((2, _RB, DH), lambda i: (0, i, 0)),
            pl.BlockSpec((_RB, DH), lambda i: (i, 0)),
            pl.BlockSpec((_RB, 1), lambda i: (i, 0)),
            pl.BlockSpec((1, DH), lambda i: (0, 0)),
            pl.BlockSpec((DH, 4 * LH), lambda i: (0, 0)),
            pl.BlockSpec((4, LH), lambda i: (0, 0)),
            pl.BlockSpec((4, LH), lambda i: (0, 0)),
        ],
        out_specs=[pl.BlockSpec((_RB, LH), lambda i: (i, 0))] * 4,
        out_shape=[jax.ShapeDtypeStruct((N, LH), jnp.float32)] * 4,
    )(accp2[:, :N, :], ys2, dinv, b2.reshape(1, DH), jnp.transpose(W_ih),
      b_ih.reshape(4, LH), b_hh.reshape(4, LH))

    wsplit = jnp.transpose(W_hh).reshape(LH, 4, LH).transpose(1, 0, 2)
    out = pl.pallas_call(
        _tc_lstm_head,
        out_shape=jax.ShapeDtypeStruct((NG, NC), jnp.float32),
        scratch_shapes=[pltpu.VMEM((N, LH), jnp.float32)],
    )(xgs[0], xgs[1], xgs[2], xgs[3], wsplit, batch.reshape(1, N), Wl,
      bl.reshape(1, NC))
    return out


# speedup vs baseline: 2.8930x; 1.1146x over previous
import functools

import jax
import jax.numpy as jnp
from jax import lax
from jax.experimental import pallas as pl
from jax.experimental.pallas import tpu as pltpu
from jax.experimental.pallas import tpu_sc as plsc

N = 10000
E = 320000
DF = 128
DH = 64
LH = 32
NC = 10
NG = 64

NPAD = 10240
CH = 125
CPW = 80
RPT = NPAD // 16

_sc_mesh = plsc.VectorSubcoreMesh(core_axis_name="c", subcore_axis_name="s")


@functools.partial(
    pl.kernel,
    mesh=_sc_mesh,
    out_type=jax.ShapeDtypeStruct((2, NPAD, 16), jnp.float32),
    scratch_types=[
        pltpu.VMEM((CPW, CH), jnp.int32),
        pltpu.VMEM((CH, 16), jnp.float32),
        pltpu.VMEM_SHARED((NPAD, 16), jnp.float32),
    ],
    compiler_params=pltpu.CompilerParams(use_tc_tiling_on_sc=False),
)
def _sc_degree(dst_hbm, zeros_hbm, out_hbm, dst_buf, ones_v, acc_sh):
    c = lax.axis_index("c")
    s = lax.axis_index("s")
    w = s * 2 + c

    def fill(i, carry):
        ones_v[i, :] = jnp.ones((16,), jnp.float32)
        return carry

    lax.fori_loop(0, CH, fill, 0)
    pltpu.sync_copy(zeros_hbm.at[pl.ds(s * RPT, RPT), :],
                    acc_sh.at[pl.ds(s * RPT, RPT), :])
    pltpu.sync_copy(dst_hbm.at[pl.ds(w * CPW, CPW), :], dst_buf)
    plsc.subcore_barrier()
    for j in range(CPW):
        pltpu.sync_copy(ones_v, acc_sh.at[dst_buf.at[j]], add=True)
    plsc.subcore_barrier()
    pltpu.sync_copy(acc_sh.at[pl.ds(s * RPT, RPT), :],
                    out_hbm.at[c, pl.ds(s * RPT, RPT), :])


@functools.partial(
    pl.kernel,
    mesh=_sc_mesh,
    out_type=jax.ShapeDtypeStruct((2, NPAD, DH), jnp.float32),
    scratch_types=[
        pltpu.VMEM((CPW, CH), jnp.int32),
        pltpu.VMEM((CPW, CH), jnp.int32),
        pltpu.VMEM((CH, DH), jnp.float32),
        pltpu.VMEM((CH, DH), jnp.float32),
        pltpu.SemaphoreType.DMA,
        pltpu.SemaphoreType.DMA,
        pltpu.VMEM_SHARED((NPAD, DH), jnp.float32),
    ],
    compiler_params=pltpu.CompilerParams(use_tc_tiling_on_sc=False),
)
def _sc_scatter(ys_hbm, src_hbm, dst_hbm, zeros_hbm, out_hbm,
                src_buf, dst_buf, rows0, rows1, sem0, sem1, acc_sh):
    c = lax.axis_index("c")
    s = lax.axis_index("s")
    w = s * 2 + c

    pltpu.sync_copy(zeros_hbm.at[pl.ds(s * RPT, RPT), :],
                    acc_sh.at[pl.ds(s * RPT, RPT), :])
    pltpu.sync_copy(src_hbm.at[pl.ds(w * CPW, CPW), :], src_buf)
    pltpu.sync_copy(dst_hbm.at[pl.ds(w * CPW, CPW), :], dst_buf)
    plsc.subcore_barrier()

    rows = (rows0, rows1)
    sems = (sem0, sem1)
    descs = [None, None]
    descs[0] = pltpu.async_copy(ys_hbm.at[src_buf.at[0]], rows0, sem0)
    for j in range(CPW):
        b = j % 2
        nb = (j + 1) % 2
        if j + 1 < CPW:
            descs[nb] = pltpu.async_copy(ys_hbm.at[src_buf.at[j + 1]],
                                         rows[nb], sems[nb])
        descs[b].wait()
        pltpu.sync_copy(rows[b], acc_sh.at[dst_buf.at[j]], add=True)
    plsc.subcore_barrier()
    pltpu.sync_copy(acc_sh.at[pl.ds(s * RPT, RPT), :],
                    out_hbm.at[c, pl.ds(s * RPT, RPT), :])


_RB = 1000


def _tc_xw1_body(x_ref, w1_ref, degt_ref, ys_ref, dinv_ref):
    deg = degt_ref[:, 0:1] + degt_ref[:, 1:2] + 1.0
    dinv = lax.rsqrt(deg)
    xw = jnp.dot(x_ref[...], w1_ref[...], preferred_element_type=jnp.float32)
    ys_ref[...] = xw * dinv
    dinv_ref[...] = dinv


def _tc_layer1(acc_ref, ys_ref, dinv_ref, b1_ref, w2_ref, ys2_ref):
    agg = acc_ref[0] + acc_ref[1] + ys_ref[...]
    dinv = dinv_ref[...]
    h1 = jnp.maximum(agg * dinv + b1_ref[...], 0.0)
    ys2_ref[...] = jnp.dot(h1, w2_ref[...],
                           preferred_element_type=jnp.float32) * dinv


def _tc_layer2(acc_ref, ys_ref, dinv_ref, b2_ref, wihT_ref, bih_ref, bhh_ref,
               xg0_ref, xg1_ref, xg2_ref, xg3_ref):
    agg = acc_ref[0] + acc_ref[1] + ys_ref[...]
    dinv = dinv_ref[...]
    h2 = jnp.maximum(agg * dinv + b2_ref[...], 0.0)
    outs = (xg0_ref, xg1_ref, xg2_ref, xg3_ref)
    for k in range(4):
        bk = bih_ref[k:k + 1, :] + bhh_ref[k:k + 1, :]
        outs[k][...] = jnp.dot(h2, wihT_ref[:, k * LH:(k + 1) * LH],
                               preferred_element_type=jnp.float32) + bk


def _sigm(z):
    return 0.5 + 0.5 * jnp.tanh(0.5 * z)


def _tc_lstm_head(xg0_ref, xg1_ref, xg2_ref, xg3_ref, wsplit_ref, batch_ref,
                  wl_ref, bl_ref, out_ref, hs_ref):
    w_i = wsplit_ref[0].astype(jnp.bfloat16)
    w_f = wsplit_ref[1].astype(jnp.bfloat16)
    w_g = wsplit_ref[2].astype(jnp.bfloat16)
    w_o = wsplit_ref[3].astype(jnp.bfloat16)

    def blockstep(k, carry):
        h, cc = carry
        x0 = xg0_ref[pl.ds(k * 8, 8), :]
        x1 = xg1_ref[pl.ds(k * 8, 8), :]
        x2 = xg2_ref[pl.ds(k * 8, 8), :]
        x3 = xg3_ref[pl.ds(k * 8, 8), :]
        hrows = []
        for j in range(8):
            hb = h.astype(jnp.bfloat16)
            i = _sigm(x0[j:j + 1, :] +
                      jnp.dot(hb, w_i, preferred_element_type=jnp.float32))
            f = _sigm(x1[j:j + 1, :] +
                      jnp.dot(hb, w_f, preferred_element_type=jnp.float32))
            gg = jnp.tanh(
                x2[j:j + 1, :] +
                jnp.dot(hb, w_g, preferred_element_type=jnp.float32))
            o = _sigm(x3[j:j + 1, :] +
                      jnp.dot(hb, w_o, preferred_element_type=jnp.float32))
            cc = f * cc + i * gg
            h = o * jnp.tanh(cc)
            hrows.append(h)
        hs_ref[pl.ds(k * 8, 8), :] = jnp.concatenate(hrows, axis=0)
        return (h, cc)

    h0 = jnp.zeros((1, LH), jnp.float32)
    c0 = jnp.zeros((1, LH), jnp.float32)
    lax.fori_loop(0, N // 8, blockstep, (h0, c0))

    sums = jnp.zeros((NG, LH), jnp.float32)
    cnt = jnp.zeros((NG, 1), jnp.float32)
    for cb in range(N // _RB):
        hsb = hs_ref[cb * _RB:(cb + 1) * _RB, :]
        bb = batch_ref[:, cb * _RB:(cb + 1) * _RB]
        ids = lax.broadcasted_iota(jnp.int32, (NG, _RB), 0)
        oh = (bb == ids).astype(jnp.float32)
        sums = sums + jnp.dot(oh, hsb, preferred_element_type=jnp.float32)
        cnt = cnt + jnp.sum(oh, axis=1, keepdims=True)
    pooled = sums / jnp.maximum(cnt, 1.0)
    logits = jnp.dot(pooled, wl_ref[...],
                     preferred_element_type=jnp.float32) + bl_ref[...]
    m = jnp.max(logits, axis=1, keepdims=True)
    lse = jnp.log(jnp.sum(jnp.exp(logits - m), axis=1, keepdims=True))
    out_ref[...] = logits - m - lse


def kernel(x, edge_index, batch, W1, b1, W2, b2, W_ih, W_hh, b_ih, b_hh,
           Wl, bl):
    src = edge_index[0].reshape(E // CH, CH)
    dst = edge_index[1].reshape(E // CH, CH)
    zeros16 = jnp.zeros((NPAD, 16), jnp.float32)
    zeros64 = jnp.zeros((NPAD, DH), jnp.float32)

    degp = _sc_degree(dst, zeros16)
    degT = jnp.transpose(degp[:, :N, 0])

    grid = (N // _RB,)
    ys1, dinv = pl.pallas_call(
        _tc_xw1_body,
        grid=grid,
        in_specs=[
            pl.BlockSpec((_RB, DF), lambda i: (i, 0)),
            pl.BlockSpec((DF, DH), lambda i: (0, 0)),
            pl.BlockSpec((_RB, 2), lambda i: (i, 0)),
        ],
        out_specs=[
            pl.BlockSpec((_RB, DH), lambda i: (i, 0)),
            pl.BlockSpec((_RB, 1), lambda i: (i, 0)),
        ],
        out_shape=[
            jax.ShapeDtypeStruct((N, DH), jnp.float32),
            jax.ShapeDtypeStruct((N, 1), jnp.float32),
        ],
    )(x, W1, degT)

    accp1 = _sc_scatter(ys1, src, dst, zeros64)

    ys2 = pl.pallas_call(
        _tc_layer1,
        grid=grid,
        in_specs=[
            pl.BlockSpec((2, _RB, DH), lambda i: (0, i, 0)),
            pl.BlockSpec((_RB, DH), lambda i: (i, 0)),
            pl.BlockSpec((_RB, 1), lambda i: (i, 0)),
            pl.BlockSpec((1, DH), lambda i: (0, 0)),
            pl.BlockSpec((DH, DH), lambda i: (0, 0)),
        ],
        out_specs=pl.BlockSpec((_RB, DH), lambda i: (i, 0)),
        out_shape=jax.ShapeDtypeStruct((N, DH), jnp.float32),
    )(accp1[:, :N, :], ys1, dinv, b1.reshape(1, DH), W2)

    accp2 = _sc_scatter(ys2, src, dst, zeros64)

    xgs = pl.pallas_call(
        _tc_layer2,
        grid=grid,
        in_specs=[
            pl.BlockSpec((2, _RB, DH), lambda i: (0, i, 0)),
            pl.BlockSpec((_RB, DH), lambda i: (i, 0)),
            pl.BlockSpec((_RB, 1), lambda i: (i, 0)),
            pl.BlockSpec((1, DH), lambda i: (0, 0)),
            pl.BlockSpec((DH, 4 * LH), lambda i: (0, 0)),
            pl.BlockSpec((4, LH), lambda i: (0, 0)),
            pl.BlockSpec((4, LH), lambda i: (0, 0)),
        ],
        out_specs=[pl.BlockSpec((_RB, LH), lambda i: (i, 0))] * 4,
        out_shape=[jax.ShapeDtypeStruct((N, LH), jnp.float32)] * 4,
    )(accp2[:, :N, :], ys2, dinv, b2.reshape(1, DH), jnp.transpose(W_ih),
      b_ih.reshape(4, LH), b_hh.reshape(4, LH))

    wsplit = jnp.transpose(W_hh).reshape(LH, 4, LH).transpose(1, 0, 2)
    out = pl.pallas_call(
        _tc_lstm_head,
        out_shape=jax.ShapeDtypeStruct((NG, NC), jnp.float32),
        scratch_shapes=[pltpu.VMEM((N, LH), jnp.float32)],
    )(xgs[0], xgs[1], xgs[2], xgs[3], wsplit, batch.reshape(1, N), Wl,
      bl.reshape(1, NC))
    return out


# speedup vs baseline: 2.8955x; 1.0009x over previous
import functools

import jax
import jax.numpy as jnp
from jax import lax
from jax.experimental import pallas as pl
from jax.experimental.pallas import tpu as pltpu
from jax.experimental.pallas import tpu_sc as plsc

N = 10000
E = 320000
DF = 128
DH = 64
LH = 32
NC = 10
NG = 64

NPAD = 10240
CH = 125
CPW = 80
RPT = NPAD // 16

_sc_mesh = plsc.VectorSubcoreMesh(core_axis_name="c", subcore_axis_name="s")


@functools.partial(
    pl.kernel,
    mesh=_sc_mesh,
    out_type=jax.ShapeDtypeStruct((2, NPAD, 16), jnp.float32),
    scratch_types=[
        pltpu.VMEM((CPW, CH), jnp.int32),
        pltpu.VMEM((CH, 16), jnp.float32),
        pltpu.VMEM_SHARED((NPAD, 16), jnp.float32),
    ],
    compiler_params=pltpu.CompilerParams(use_tc_tiling_on_sc=False),
)
def _sc_degree(dst_hbm, zeros_hbm, out_hbm, dst_buf, ones_v, acc_sh):
    c = lax.axis_index("c")
    s = lax.axis_index("s")
    w = s * 2 + c

    def fill(i, carry):
        ones_v[i, :] = jnp.ones((16,), jnp.float32)
        return carry

    lax.fori_loop(0, CH, fill, 0)
    pltpu.sync_copy(zeros_hbm.at[pl.ds(s * RPT, RPT), :],
                    acc_sh.at[pl.ds(s * RPT, RPT), :])
    pltpu.sync_copy(dst_hbm.at[pl.ds(w * CPW, CPW), :], dst_buf)
    plsc.subcore_barrier()
    for j in range(CPW):
        pltpu.sync_copy(ones_v, acc_sh.at[dst_buf.at[j]], add=True)
    plsc.subcore_barrier()
    pltpu.sync_copy(acc_sh.at[pl.ds(s * RPT, RPT), :],
                    out_hbm.at[c, pl.ds(s * RPT, RPT), :])


@functools.partial(
    pl.kernel,
    mesh=_sc_mesh,
    out_type=jax.ShapeDtypeStruct((2, NPAD, DH), jnp.float32),
    scratch_types=[
        pltpu.VMEM((CPW, CH), jnp.int32),
        pltpu.VMEM((CPW, CH), jnp.int32),
        pltpu.VMEM((CH, DH), jnp.float32),
        pltpu.VMEM((CH, DH), jnp.float32),
        pltpu.SemaphoreType.DMA,
        pltpu.SemaphoreType.DMA,
        pltpu.VMEM_SHARED((NPAD, DH), jnp.float32),
    ],
    compiler_params=pltpu.CompilerParams(use_tc_tiling_on_sc=False),
)
def _sc_scatter(ys_hbm, src_hbm, dst_hbm, zeros_hbm, out_hbm,
                src_buf, dst_buf, rows0, rows1, sem0, sem1, acc_sh):
    c = lax.axis_index("c")
    s = lax.axis_index("s")
    w = s * 2 + c

    pltpu.sync_copy(zeros_hbm.at[pl.ds(s * RPT, RPT), :],
                    acc_sh.at[pl.ds(s * RPT, RPT), :])
    pltpu.sync_copy(src_hbm.at[pl.ds(w * CPW, CPW), :], src_buf)
    pltpu.sync_copy(dst_hbm.at[pl.ds(w * CPW, CPW), :], dst_buf)
    plsc.subcore_barrier()

    rows = (rows0, rows1)
    sems = (sem0, sem1)
    descs = [None, None]
    descs[0] = pltpu.async_copy(ys_hbm.at[src_buf.at[0]], rows0, sem0)
    for j in range(CPW):
        b = j % 2
        nb = (j + 1) % 2
        if j + 1 < CPW:
            descs[nb] = pltpu.async_copy(ys_hbm.at[src_buf.at[j + 1]],
                                         rows[nb], sems[nb])
        descs[b].wait()
        pltpu.sync_copy(rows[b], acc_sh.at[dst_buf.at[j]], add=True)
    plsc.subcore_barrier()
    pltpu.sync_copy(acc_sh.at[pl.ds(s * RPT, RPT), :],
                    out_hbm.at[c, pl.ds(s * RPT, RPT), :])


_RB = 1000


def _tc_xw1_body(x_ref, w1_ref, degt_ref, ys_ref, dinv_ref):
    deg = degt_ref[:, 0:1] + degt_ref[:, 1:2] + 1.0
    dinv = lax.rsqrt(deg)
    xw = jnp.dot(x_ref[...], w1_ref[...], preferred_element_type=jnp.float32)
    ys_ref[...] = xw * dinv
    dinv_ref[...] = dinv


def _tc_layer1(acc_ref, ys_ref, dinv_ref, b1_ref, w2_ref, ys2_ref):
    agg = acc_ref[0] + acc_ref[1] + ys_ref[...]
    dinv = dinv_ref[...]
    h1 = jnp.maximum(agg * dinv + b1_ref[...], 0.0)
    ys2_ref[...] = jnp.dot(h1, w2_ref[...],
                           preferred_element_type=jnp.float32) * dinv


def _tc_layer2(acc_ref, ys_ref, dinv_ref, b2_ref, wihT_ref, bih_ref, bhh_ref,
               xg0_ref, xg1_ref, xg2_ref, xg3_ref):
    agg = acc_ref[0] + acc_ref[1] + ys_ref[...]
    dinv = dinv_ref[...]
    h2 = jnp.maximum(agg * dinv + b2_ref[...], 0.0)
    outs = (xg0_ref, xg1_ref, xg2_ref, xg3_ref)
    for k in range(4):
        bk = bih_ref[k:k + 1, :] + bhh_ref[k:k + 1, :]
        outs[k][...] = jnp.dot(h2, wihT_ref[:, k * LH:(k + 1) * LH],
                               preferred_element_type=jnp.float32) + bk


def _sigm(z):
    return 0.5 + 0.5 * jnp.tanh(0.5 * z)


def _tc_lstm_head(xg0_ref, xg1_ref, xg2_ref, xg3_ref, wsplit_ref, batch_ref,
                  wl_ref, bl_ref, out_ref, hs_ref):
    w_i = wsplit_ref[0].astype(jnp.bfloat16)
    w_f = wsplit_ref[1].astype(jnp.bfloat16)
    w_g = wsplit_ref[2].astype(jnp.bfloat16)
    w_o = wsplit_ref[3].astype(jnp.bfloat16)

    def blockstep(k, carry):
        h, cc = carry
        x0 = xg0_ref[pl.ds(k * 16, 16), :]
        x1 = xg1_ref[pl.ds(k * 16, 16), :]
        x2 = xg2_ref[pl.ds(k * 16, 16), :]
        x3 = xg3_ref[pl.ds(k * 16, 16), :]
        hrows = []
        for j in range(16):
            hb = h.astype(jnp.bfloat16)
            i = _sigm(x0[j:j + 1, :] +
                      jnp.dot(hb, w_i, preferred_element_type=jnp.float32))
            f = _sigm(x1[j:j + 1, :] +
                      jnp.dot(hb, w_f, preferred_element_type=jnp.float32))
            gg = jnp.tanh(
                x2[j:j + 1, :] +
                jnp.dot(hb, w_g, preferred_element_type=jnp.float32))
            o = _sigm(x3[j:j + 1, :] +
                      jnp.dot(hb, w_o, preferred_element_type=jnp.float32))
            cc = f * cc + i * gg
            h = o * jnp.tanh(cc)
            hrows.append(h)
        hs_ref[pl.ds(k * 16, 16), :] = jnp.concatenate(hrows, axis=0)
        return (h, cc)

    h0 = jnp.zeros((1, LH), jnp.float32)
    c0 = jnp.zeros((1, LH), jnp.float32)
    lax.fori_loop(0, N // 16, blockstep, (h0, c0))

    sums = jnp.zeros((NG, LH), jnp.float32)
    cnt = jnp.zeros((NG, 1), jnp.float32)
    for cb in range(N // _RB):
        hsb = hs_ref[cb * _RB:(cb + 1) * _RB, :]
        bb = batch_ref[:, cb * _RB:(cb + 1) * _RB]
        ids = lax.broadcasted_iota(jnp.int32, (NG, _RB), 0)
        oh = (bb == ids).astype(jnp.float32)
        sums = sums + jnp.dot(oh, hsb, preferred_element_type=jnp.float32)
        cnt = cnt + jnp.sum(oh, axis=1, keepdims=True)
    pooled = sums / jnp.maximum(cnt, 1.0)
    logits = jnp.dot(pooled, wl_ref[...],
                     preferred_element_type=jnp.float32) + bl_ref[...]
    m = jnp.max(logits, axis=1, keepdims=True)
    lse = jnp.log(jnp.sum(jnp.exp(logits - m), axis=1, keepdims=True))
    out_ref[...] = logits - m - lse


def kernel(x, edge_index, batch, W1, b1, W2, b2, W_ih, W_hh, b_ih, b_hh,
           Wl, bl):
    src = edge_index[0].reshape(E // CH, CH)
    dst = edge_index[1].reshape(E // CH, CH)
    zeros16 = jnp.zeros((NPAD, 16), jnp.float32)
    zeros64 = jnp.zeros((NPAD, DH), jnp.float32)

    degp = _sc_degree(dst, zeros16)
    degT = jnp.transpose(degp[:, :N, 0])

    grid = (N // _RB,)
    ys1, dinv = pl.pallas_call(
        _tc_xw1_body,
        grid=grid,
        in_specs=[
            pl.BlockSpec((_RB, DF), lambda i: (i, 0)),
            pl.BlockSpec((DF, DH), lambda i: (0, 0)),
            pl.BlockSpec((_RB, 2), lambda i: (i, 0)),
        ],
        out_specs=[
            pl.BlockSpec((_RB, DH), lambda i: (i, 0)),
            pl.BlockSpec((_RB, 1), lambda i: (i, 0)),
        ],
        out_shape=[
            jax.ShapeDtypeStruct((N, DH), jnp.float32),
            jax.ShapeDtypeStruct((N, 1), jnp.float32),
        ],
    )(x, W1, degT)

    accp1 = _sc_scatter(ys1, src, dst, zeros64)

    ys2 = pl.pallas_call(
        _tc_layer1,
        grid=grid,
        in_specs=[
            pl.BlockSpec((2, _RB, DH), lambda i: (0, i, 0)),
            pl.BlockSpec((_RB, DH), lambda i: (i, 0)),
            pl.BlockSpec((_RB, 1), lambda i: (i, 0)),
            pl.BlockSpec((1, DH), lambda i: (0, 0)),
            pl.BlockSpec((DH, DH), lambda i: (0, 0)),
        ],
        out_specs=pl.BlockSpec((_RB, DH), lambda i: (i, 0)),
        out_shape=jax.ShapeDtypeStruct((N, DH), jnp.float32),
    )(accp1[:, :N, :], ys1, dinv, b1.reshape(1, DH), W2)

    accp2 = _sc_scatter(ys2, src, dst, zeros64)

    xgs = pl.pallas_call(
        _tc_layer2,
        grid=grid,
        in_specs=[
            pl.BlockSpec((2, _RB, DH), lambda i: (0, i, 0)),
            pl.BlockSpec((_RB, DH), lambda i: (i, 0)),
            pl.BlockSpec((_RB, 1), lambda i: (i, 0)),
            pl.BlockSpec((1, DH), lambda i: (0, 0)),
            pl.BlockSpec((DH, 4 * LH), lambda i: (0, 0)),
            pl.BlockSpec((4, LH), lambda i: (0, 0)),
            pl.BlockSpec((4, LH), lambda i: (0, 0)),
        ],
        out_specs=[pl.BlockSpec((_RB, LH), lambda i: (i, 0))] * 4,
        out_shape=[jax.ShapeDtypeStruct((N, LH), jnp.float32)] * 4,
    )(accp2[:, :N, :], ys2, dinv, b2.reshape(1, DH), jnp.transpose(W_ih),
      b_ih.reshape(4, LH), b_hh.reshape(4, LH))

    wsplit = jnp.transpose(W_hh).reshape(LH, 4, LH).transpose(1, 0, 2)
    out = pl.pallas_call(
        _tc_lstm_head,
        out_shape=jax.ShapeDtypeStruct((NG, NC), jnp.float32),
        scratch_shapes=[pltpu.VMEM((N, LH), jnp.float32)],
    )(xgs[0], xgs[1], xgs[2], xgs[3], wsplit, batch.reshape(1, N), Wl,
      bl.reshape(1, NC))
    return out
